# baseline (speedup 1.0000x reference)
import jax
import jax.numpy as jnp
from jax import lax
from jax.experimental import pallas as pl
from jax.experimental.pallas import tpu as pltpu

B, L, D = 16, 4096, 1024
CH = 256
NCH = (L + 1 + CH - 1) // CH


def _body(lens_ref, x_ref, first_ref, tok_ref, ts_ref,
          out_x_ref, out_ts_ref, out_len_ref):
    b = pl.program_id(0)
    c = pl.program_id(1)
    last = lens_ref[b]

    rows = lax.broadcasted_iota(jnp.int32, (CH, 1), 0) + c * CH
    y = jnp.where(rows == last, tok_ref[...], x_ref[0])
    y = jnp.where(rows == L, first_ref[0, 0:1], y)
    out_x_ref[0] = y

    @pl.when(c == 0)
    def _ts():
        ts = ts_ref[0]
        cols = lax.broadcasted_iota(jnp.int32, (1, L), 1)
        last_m1 = jnp.maximum(last - 1, 0)
        last_ts = jnp.sum(jnp.where(cols == last_m1, ts, 0.0))
        out_ts_ref[0, :, :L] = jnp.where(cols == last, last_ts, ts)
        out_ts_ref[0, :, L:L + 1] = ts[:, 0:1]

    @pl.when((b == 0) & (c == 0))
    def _len():
        for i in range(B):
            out_len_ref[i] = lens_ref[i] + 1


def kernel(x_payload, timestamps, seq_lens, token):
    seq_lens = seq_lens.astype(jnp.int32)
    token2 = token.reshape(1, D)

    grid_spec = pltpu.PrefetchScalarGridSpec(
        num_scalar_prefetch=1,
        grid=(B, NCH),
        in_specs=[
            pl.BlockSpec((1, CH, D),
                         lambda b, c, lens: (b, jnp.minimum(c, NCH - 2), 0)),
            pl.BlockSpec((1, 8, D), lambda b, c, lens: (b, 0, 0)),
            pl.BlockSpec((1, D), lambda b, c, lens: (0, 0)),
            pl.BlockSpec((1, 1, L), lambda b, c, lens: (b, 0, 0)),
        ],
        out_specs=[
            pl.BlockSpec((1, CH, D), lambda b, c, lens: (b, c, 0)),
            pl.BlockSpec((1, 1, L + 1), lambda b, c, lens: (b, 0, 0)),
            pl.BlockSpec(memory_space=pltpu.SMEM),
        ],
    )
    new_x, new_ts, new_len = pl.pallas_call(
        _body,
        grid_spec=grid_spec,
        out_shape=[
            jax.ShapeDtypeStruct((B, L + 1, D), x_payload.dtype),
            jax.ShapeDtypeStruct((B, 1, L + 1), timestamps.dtype),
            jax.ShapeDtypeStruct((B,), jnp.int32),
        ],
        compiler_params=pltpu.CompilerParams(
            dimension_semantics=("parallel", "arbitrary"),
        ),
    )(seq_lens, x_payload, x_payload, token2,
      timestamps.reshape(B, 1, L))
    return new_x, new_len, new_ts.reshape(B, L + 1), new_len
